# Initial kernel scaffold; baseline (speedup 1.0000x reference)
import jax
import jax.numpy as jnp
from jax.experimental import pallas as pl


def kernel(embeddings, mom_embeddings, k, mask, warmup):
    raise NotImplementedError("write your pallas kernel here")


# speedup vs baseline: 25.5528x; 25.5528x over previous
import numpy as np

import jax
import jax.numpy as jnp
from jax.experimental import pallas as pl

_C, _H, _W = 128, 84, 84
_N = _H * _W
_NUM_NEG = 64
_PROX = 40
_SPAN = _H - 2 * _PROX
_INV_TAU = 10.0
_ALPHA = 1.0


def _build_counts() -> np.ndarray:
    rng = np.random.default_rng(0)
    off_r = rng.integers(_PROX, _H - _PROX, size=(_N, _NUM_NEG))
    off_c = rng.integers(_PROX, _W - _PROX, size=(_N, _NUM_NEG))
    o = (off_r - _PROX) * _SPAN + (off_c - _PROX)
    flat = np.arange(_N)[:, None] * (_SPAN * _SPAN) + o
    cnt = np.bincount(flat.ravel(), minlength=_N * _SPAN * _SPAN)
    return cnt.reshape(_N, _SPAN * _SPAN).T.reshape(
        _SPAN * _SPAN, _H, _W).astype(np.float32)


_COUNTS = _build_counts()


def _shift2(x, dr, dc):
    x = jnp.concatenate([x[:, dr:, :], x[:, :dr, :]], axis=1)
    x = jnp.concatenate([x[:, :, dc:], x[:, :, :dc]], axis=2)
    return x


_CB = 16


def _loss_kernel(emb_ref, mom_ref, mask_ref, cnt_ref, out_ref):
    def body(ck, acc):
        e = emb_ref[pl.ds(ck * _CB, _CB)]
        m = mom_ref[pl.ds(ck * _CB, _CB)]
        parts = [jnp.sum(e * m, axis=0)]
        for o in range(_SPAN * _SPAN):
            dr = _PROX + o // _SPAN
            dc = _PROX + o % _SPAN
            parts.append(jnp.sum(e * _shift2(m, dr, dc), axis=0))
        return acc + jnp.stack(parts)

    sims = jax.lax.fori_loop(
        0, _C // _CB, body,
        jnp.zeros((1 + _SPAN * _SPAN, _H, _W), jnp.float32))

    msk = mask_ref[...]
    p = (1.0 + sims[0]) * _INV_TAU
    s = jnp.exp(p) + jnp.sum(
        cnt_ref[...] * jnp.exp((1.0 + sims[1:]) * _INV_TAU), axis=0)

    a = jnp.sum(jnp.log(s))
    b = jnp.sum(msk)
    c = jnp.sum(p * msk)
    n = jnp.float32(_N)
    loss = _ALPHA * (a * b - n * c) / (n * n)
    out_ref[...] = loss[None, None]


def kernel(embeddings, mom_embeddings, k, mask, warmup):
    del k, warmup
    counts = jnp.asarray(_COUNTS)
    out = pl.pallas_call(
        _loss_kernel,
        out_shape=jax.ShapeDtypeStruct((1, 1), jnp.float32),
    )(embeddings.astype(jnp.float32), mom_embeddings.astype(jnp.float32),
      mask.astype(jnp.float32), counts)
    return out[0, 0]


# speedup vs baseline: 44.3454x; 1.7354x over previous
import numpy as np

import jax
import jax.numpy as jnp
from jax.experimental import pallas as pl

_C, _H, _W = 128, 84, 84
_N = _H * _W
_NUM_NEG = 64
_PROX = 40
_SPAN = _H - 2 * _PROX
_INV_TAU = 10.0
_ALPHA = 1.0


def _build_counts() -> np.ndarray:
    rng = np.random.default_rng(0)
    off_r = rng.integers(_PROX, _H - _PROX, size=(_N, _NUM_NEG))
    off_c = rng.integers(_PROX, _W - _PROX, size=(_N, _NUM_NEG))
    o = (off_r - _PROX) * _SPAN + (off_c - _PROX)
    flat = np.arange(_N)[:, None] * (_SPAN * _SPAN) + o
    cnt = np.bincount(flat.ravel(), minlength=_N * _SPAN * _SPAN)
    return cnt.reshape(_N, _SPAN * _SPAN).T.reshape(
        _SPAN * _SPAN, _H, _W).astype(np.float32)


_COUNTS = _build_counts()


def _shift2(x, dr, dc):
    x = jnp.concatenate([x[:, dr:, :], x[:, :dr, :]], axis=1)
    x = jnp.concatenate([x[:, :, dc:], x[:, :, :dc]], axis=2)
    return x


_CB = 16


def _loss_kernel(emb_ref, mom_ref, mask_ref, cnt_ref, out_ref):
    def body(ck, acc):
        e = emb_ref[pl.ds(ck * _CB, _CB)]
        m = mom_ref[pl.ds(ck * _CB, _CB)]
        parts = [jnp.sum(e * m, axis=0)]
        mc = [jnp.concatenate([m[:, :, dc:], m[:, :, :dc]], axis=2)
              for dc in range(_PROX, _PROX + _SPAN)]
        for o in range(_SPAN * _SPAN):
            dr = _PROX + o // _SPAN
            mcd = mc[o % _SPAN]
            ms = jnp.concatenate([mcd[:, dr:, :], mcd[:, :dr, :]], axis=1)
            parts.append(jnp.sum(e * ms, axis=0))
        return acc + jnp.stack(parts)

    sims = jax.lax.fori_loop(
        0, _C // _CB, body,
        jnp.zeros((1 + _SPAN * _SPAN, _H, _W), jnp.float32))

    msk = mask_ref[...]
    p = (1.0 + sims[0]) * _INV_TAU
    s = jnp.exp(p) + jnp.sum(
        cnt_ref[...] * jnp.exp((1.0 + sims[1:]) * _INV_TAU), axis=0)

    a = jnp.sum(jnp.log(s))
    b = jnp.sum(msk)
    c = jnp.sum(p * msk)
    n = jnp.float32(_N)
    loss = _ALPHA * (a * b - n * c) / (n * n)
    out_ref[...] = loss[None, None]


def kernel(embeddings, mom_embeddings, k, mask, warmup):
    del k, warmup
    counts = jnp.asarray(_COUNTS)
    out = pl.pallas_call(
        _loss_kernel,
        out_shape=jax.ShapeDtypeStruct((1, 1), jnp.float32),
    )(embeddings.astype(jnp.float32), mom_embeddings.astype(jnp.float32),
      mask.astype(jnp.float32), counts)
    return out[0, 0]
